# baseline (speedup 1.0000x reference)
import functools

import jax
import jax.numpy as jnp
from jax import lax
from jax.experimental import pallas as pl
from jax.experimental.pallas import tpu as pltpu
from jax.experimental.pallas import tpu_sc as plsc

VOCAB = 32000
D_MODEL = 4096
NUM_DEV = 8
D_SLICE = D_MODEL // NUM_DEV

_NC = 2
_NS = 16
_NW = _NC * _NS

BATCH = 2
SEQ = 8192
TOKENS = BATCH * SEQ
ROWS = NUM_DEV * TOKENS
PT = ROWS // _NW
R = 64
NCH = PT // R
NBUF = 3
NFULL = (NCH // NBUF) * NBUF


def _body(table, idx, out, xv, bufs, gsems, wsems):
    c = lax.axis_index("c")
    s = lax.axis_index("s")
    wid = s * _NC + c
    base = wid * PT
    d = base // TOKENS
    tok = base - d * TOKENS
    bb = tok // SEQ
    sq = tok - bb * SEQ
    col = d * D_SLICE

    pltpu.sync_copy(idx.at[bb, pl.ds(sq, PT)], xv)

    def gather(i, b):
        return pltpu.make_async_copy(
            table.at[xv.at[pl.ds(i * R, R)], pl.ds(col, D_SLICE)],
            bufs[b], gsems[b],
        )

    def write(i, b):
        return pltpu.make_async_copy(
            bufs[b], out.at[d, bb, pl.ds(sq + i * R, R)], wsems[b]
        )

    for b in range(NBUF - 1):
        gather(b, b).start()

    def ring(i, carry):
        c0 = i * NBUF
        for b in range(NBUF):
            cc = c0 + b
            gather(cc, b).wait()
            write(cc, b).start()
            nb = (b + NBUF - 1) % NBUF

            @pl.when(cc + NBUF - 1 < NCH)
            def _(cc=cc, b=b, nb=nb):
                @pl.when(cc >= 1)
                def _():
                    write(cc - 1, nb).wait()

                gather(cc + NBUF - 1, nb).start()

        return carry

    lax.fori_loop(0, NFULL // NBUF, ring, 0)

    for cc in range(NFULL, NCH):
        b = cc % NBUF
        gather(cc, b).wait()
        write(cc, b).start()

    for b in range(NBUF):
        write(NCH - NBUF + b, (NCH - NBUF + b) % NBUF).wait()


@functools.partial(
    pl.kernel,
    out_type=jax.ShapeDtypeStruct((NUM_DEV, BATCH, SEQ, D_SLICE), jnp.float32),
    mesh=plsc.VectorSubcoreMesh(core_axis_name="c", subcore_axis_name="s"),
    scratch_types=[
        pltpu.VMEM((PT,), jnp.int32),
        [pltpu.VMEM((R, D_SLICE), jnp.float32) for _ in range(NBUF)],
        [pltpu.SemaphoreType.DMA for _ in range(NBUF)],
        [pltpu.SemaphoreType.DMA for _ in range(NBUF)],
    ],
)
def _emb_gather(table, idx, out, xv, bufs, gsems, wsems):
    _body(table, idx, out, xv, bufs, gsems, wsems)


def kernel(x, emb_weight):
    return _emb_gather(emb_weight, x)


# speedup vs baseline: 1.0008x; 1.0008x over previous
import functools

import jax
import jax.numpy as jnp
from jax import lax
from jax.experimental import pallas as pl
from jax.experimental.pallas import tpu as pltpu
from jax.experimental.pallas import tpu_sc as plsc

VOCAB = 32000
D_MODEL = 4096
NUM_DEV = 8
D_SLICE = D_MODEL // NUM_DEV

_NC = 2
_NS = 16
_NW = _NC * _NS

BATCH = 2
SEQ = 8192
TOKENS = BATCH * SEQ
PT = TOKENS // _NW
R = 8
NCH = PT // R
NBUF = 3
NFULL = (NCH // NBUF) * NBUF


def _body(table, idx, out, xv, bufs, gsems, wsems):
    c = lax.axis_index("c")
    s = lax.axis_index("s")
    wid = s * _NC + c
    tok = wid * PT
    bb = tok // SEQ
    sq = tok - bb * SEQ

    pltpu.sync_copy(idx.at[bb, pl.ds(sq, PT)], xv)

    def gather(i, b):
        return pltpu.make_async_copy(
            table.at[xv.at[pl.ds(i * R, R)]], bufs[b], gsems[b]
        )

    def write(i, b, d):
        return pltpu.make_async_copy(
            bufs[b].at[:, pl.ds(d * D_SLICE, D_SLICE)],
            out.at[d, bb, pl.ds(sq + i * R, R)],
            wsems[b],
        )

    def write_all(i, b):
        for d in range(NUM_DEV):
            write(i, b, d).start()

    def wait_writes(i, b):
        for d in range(NUM_DEV):
            write(i, b, d).wait()

    for b in range(NBUF - 1):
        gather(b, b).start()

    def ring(i, carry):
        c0 = i * NBUF
        for b in range(NBUF):
            cc = c0 + b
            gather(cc, b).wait()
            write_all(cc, b)
            nb = (b + NBUF - 1) % NBUF

            @pl.when(cc + NBUF - 1 < NCH)
            def _(cc=cc, b=b, nb=nb):
                @pl.when(cc >= 1)
                def _():
                    wait_writes(cc - 1, nb)

                gather(cc + NBUF - 1, nb).start()

        return carry

    lax.fori_loop(0, NFULL // NBUF, ring, 0)

    for cc in range(NFULL, NCH):
        b = cc % NBUF
        gather(cc, b).wait()
        write_all(cc, b)

    for b in range(NBUF):
        wait_writes(NCH - NBUF + b, (NCH - NBUF + b) % NBUF)


@functools.partial(
    pl.kernel,
    out_type=jax.ShapeDtypeStruct((NUM_DEV, BATCH, SEQ, D_SLICE), jnp.float32),
    mesh=plsc.VectorSubcoreMesh(core_axis_name="c", subcore_axis_name="s"),
    scratch_types=[
        pltpu.VMEM((PT,), jnp.int32),
        [pltpu.VMEM((R, D_MODEL), jnp.float32) for _ in range(NBUF)],
        [pltpu.SemaphoreType.DMA for _ in range(NBUF)],
        [pltpu.SemaphoreType.DMA for _ in range(NBUF)],
    ],
)
def _emb_gather(table, idx, out, xv, bufs, gsems, wsems):
    _body(table, idx, out, xv, bufs, gsems, wsems)


def kernel(x, emb_weight):
    return _emb_gather(emb_weight, x)
